# baseline (speedup 1.0000x reference)
import functools

import jax
import jax.numpy as jnp
from jax.experimental import pallas as pl
from jax.experimental.pallas import tpu as pltpu

NC = 80


def _corners(cx, cy, w, h):
    x1 = cx - w / 2
    y1 = cy - h / 2
    x2 = cx + w / 2
    y2 = cy + h / 2
    return y1, x1, y2, x2


def _inter_union(g_cx, g_cy, g_w, g_h, p_cx, p_cy, p_w, p_h):
    gy1, gx1, gy2, gx2 = _corners(g_cx, g_cy, g_w, g_h)
    py1, px1, py2, px2 = _corners(p_cx, p_cy, p_w, p_h)
    in_ymin = jnp.maximum(gy1, py1)
    in_xmin = jnp.maximum(gx1, px1)
    in_ymax = jnp.minimum(gy2, py2)
    in_xmax = jnp.minimum(gx2, px2)
    in_w = jnp.maximum(0.0, in_xmax - in_xmin)
    in_h = jnp.maximum(0.0, in_ymax - in_ymin)
    inter = in_w * in_h
    areas = (g_w * g_h) + (p_w * p_h)
    union = areas - inter
    return inter, union


def _split_gt(gt):
    g_cx = gt[:, 0:1]
    g_cy = gt[:, 1:2]
    g_w = gt[:, 2:3]
    g_h = gt[:, 3:4]
    return g_cx, g_cy, g_w, g_h


def _argmax_kernel(gt_ref, px_ref, py_ref, pw_ref, ph_ref, best_ref, m_ref,
                   a_ref, *, blk, num_pr, pb_steps):
    pb = pl.program_id(1)
    gt = gt_ref[0]
    ng = gt.shape[0]
    g_cx, g_cy, g_w, g_h = _split_gt(gt)
    valid = g_cx != -1.0
    inter, union = _inter_union(g_cx, g_cy, g_w, g_h,
                                px_ref[0], py_ref[0], pw_ref[0], ph_ref[0])
    iou = inter / (union + 1e-5)
    iou = jnp.where(valid, iou, 0.0)

    @pl.when(pb == 0)
    def _():
        m_ref[...] = jnp.full((ng, 1), -1.0, jnp.float32)
        a_ref[...] = jnp.zeros((ng, 1), jnp.int32)

    bmax = jnp.max(iou, axis=1, keepdims=True)
    pidx = jax.lax.broadcasted_iota(jnp.int32, iou.shape, 1) + pb * blk
    barg = jnp.min(jnp.where(iou == bmax, pidx, num_pr), axis=1, keepdims=True)
    better = bmax > m_ref[...]
    a_ref[...] = jnp.where(better, barg, a_ref[...])
    m_ref[...] = jnp.where(better, bmax, m_ref[...])

    @pl.when(pb == pb_steps - 1)
    def _():
        best_ref[0] = a_ref[...]


def _assign_kernel(gt_ref, px_ref, py_ref, pw_ref, ph_ref,
                   bx_ref, by_ref, bw_ref, bh_ref, best_ref, packed_ref,
                   *, blk):
    pb = pl.program_id(1)
    gt = gt_ref[0]
    ng = gt.shape[0]
    g_cx, g_cy, g_w, g_h = _split_gt(gt)
    g_cls = gt[:, 4:5]
    g_conf = gt[:, 5:6]
    valid = g_cx != -1.0

    inter, union = _inter_union(g_cx, g_cy, g_w, g_h,
                                px_ref[0], py_ref[0], pw_ref[0], ph_ref[0])
    ue = union + 1e-5
    thr = (inter >= 0.5 * ue) & valid
    ign = (inter >= 0.4 * ue) & (inter < 0.5 * ue) & valid

    pidx = jax.lax.broadcasted_iota(jnp.int32, inter.shape, 1) + pb * blk
    best = best_ref[0]
    is_best = (pidx == best) & (g_conf > 0.0)

    g_iota = jax.lax.broadcasted_iota(jnp.int32, inter.shape, 0)
    score = jnp.where(is_best, g_iota + ng, jnp.where(thr, g_iota, -1))
    smax = jnp.max(score, axis=0, keepdims=True)
    matched = smax >= 0
    sel = (score == smax) & matched
    cls_true = jnp.sum(jnp.where(sel, g_cls, 0.0), axis=0, keepdims=True)
    cls_true = jnp.where(matched, cls_true, float(NC))

    b_cx = bx_ref[0]
    b_cy = by_ref[0]
    b_w = bw_ref[0]
    b_h = bh_ref[0]
    cnt = thr.astype(jnp.float32) + is_best.astype(jnp.float32)
    lgw = jnp.log(jnp.where(valid, g_w, 1.0))
    lgh = jnp.log(jnp.where(valid, g_h, 1.0))
    s_cnt = jnp.sum(cnt, axis=0, keepdims=True)
    s_cx = jnp.sum(cnt * g_cx, axis=0, keepdims=True)
    s_cy = jnp.sum(cnt * g_cy, axis=0, keepdims=True)
    s_lw = jnp.sum(cnt * lgw, axis=0, keepdims=True)
    s_lh = jnp.sum(cnt * lgh, axis=0, keepdims=True)
    l0 = (s_cx - b_cx * s_cnt) / b_w
    l1 = (s_cy - b_cy * s_cnt) / b_h
    l2 = s_lw - s_cnt * jnp.log(b_w)
    l3 = s_lh - s_cnt * jnp.log(b_h)

    bg = (cls_true == float(NC)).astype(jnp.float32)
    ignore_any = jnp.max(ign.astype(jnp.int32), axis=0, keepdims=True) > 0
    amask = jnp.where(ignore_any, -1.0, bg)

    zeros2 = jnp.zeros((2,) + cls_true.shape[1:], jnp.float32)
    packed = jnp.concatenate(
        [cls_true, l0, l1, l2, l3, amask, zeros2], axis=0)
    packed_ref[0] = jnp.transpose(packed)


def _emit_kernel(packed_ref, cls_ref, loc_ref, msk_ref):
    d = packed_ref[0]
    cls_true = d[:, 0:1].astype(jnp.int32)
    c_iota = jax.lax.broadcasted_iota(jnp.int32, (d.shape[0], NC), 1)
    cls_ref[0] = (c_iota == cls_true).astype(jnp.float32)
    loc_ref[0] = d[:, 1:5]
    msk_ref[0] = d[:, 5:6]


@jax.jit
def kernel(gt_boxes, pr_boxes):
    B, NG, _ = gt_boxes.shape
    _, NP, _ = pr_boxes.shape
    blk = 2048
    npad = -NP % blk
    NPP = NP + npad
    pb_steps = NPP // blk

    comp = jnp.pad(pr_boxes, ((0, 0), (0, npad), (0, 0)))
    px = comp[:, :, 0].reshape(B, 1, NPP)
    py = comp[:, :, 1].reshape(B, 1, NPP)
    pw = comp[:, :, 2].reshape(B, 1, NPP)
    ph = comp[:, :, 3].reshape(B, 1, NPP)

    row_spec = lambda f: pl.BlockSpec((1, 1, blk), f)
    b0 = lambda b, p: (0, 0, p)
    bb = lambda b, p: (b, 0, p)

    best = pl.pallas_call(
        functools.partial(_argmax_kernel, blk=blk, num_pr=NPP,
                          pb_steps=pb_steps),
        grid=(B, pb_steps),
        in_specs=[
            pl.BlockSpec((1, NG, 6), lambda b, p: (b, 0, 0)),
            row_spec(b0), row_spec(b0), row_spec(b0), row_spec(b0),
        ],
        out_specs=pl.BlockSpec((1, NG, 1), lambda b, p: (b, 0, 0)),
        out_shape=jax.ShapeDtypeStruct((B, NG, 1), jnp.int32),
        scratch_shapes=[
            pltpu.VMEM((NG, 1), jnp.float32),
            pltpu.VMEM((NG, 1), jnp.int32),
        ],
    )(gt_boxes, px, py, pw, ph)

    packed = pl.pallas_call(
        functools.partial(_assign_kernel, blk=blk),
        grid=(B, pb_steps),
        in_specs=[
            pl.BlockSpec((1, NG, 6), lambda b, p: (b, 0, 0)),
            row_spec(b0), row_spec(b0), row_spec(b0), row_spec(b0),
            row_spec(bb), row_spec(bb), row_spec(bb), row_spec(bb),
            pl.BlockSpec((1, NG, 1), lambda b, p: (b, 0, 0)),
        ],
        out_specs=pl.BlockSpec((1, blk, 8), lambda b, p: (b, p, 0)),
        out_shape=jax.ShapeDtypeStruct((B, NPP, 8), jnp.float32),
    )(gt_boxes, px, py, pw, ph, px, py, pw, ph, best)

    blk3 = 4000
    cls_out, loc_true, amask = pl.pallas_call(
        _emit_kernel,
        grid=(B, NP // blk3),
        in_specs=[pl.BlockSpec((1, blk3, 8), lambda b, p: (b, p, 0))],
        out_specs=[
            pl.BlockSpec((1, blk3, NC), lambda b, p: (b, p, 0)),
            pl.BlockSpec((1, blk3, 4), lambda b, p: (b, p, 0)),
            pl.BlockSpec((1, blk3, 1), lambda b, p: (b, p, 0)),
        ],
        out_shape=[
            jax.ShapeDtypeStruct((B, NP, NC), jnp.float32),
            jax.ShapeDtypeStruct((B, NP, 4), jnp.float32),
            jax.ShapeDtypeStruct((B, NP, 1), jnp.float32),
        ],
    )(packed)

    return (cls_out, loc_true, amask)


# speedup vs baseline: 1.0836x; 1.0836x over previous
import functools

import jax
import jax.numpy as jnp
from jax.experimental import pallas as pl
from jax.experimental.pallas import tpu as pltpu

NC = 80


def _corners(cx, cy, w, h):
    x1 = cx - w / 2
    y1 = cy - h / 2
    x2 = cx + w / 2
    y2 = cy + h / 2
    return y1, x1, y2, x2


def _inter_union(g_cx, g_cy, g_w, g_h, p_cx, p_cy, p_w, p_h):
    gy1, gx1, gy2, gx2 = _corners(g_cx, g_cy, g_w, g_h)
    py1, px1, py2, px2 = _corners(p_cx, p_cy, p_w, p_h)
    in_ymin = jnp.maximum(gy1, py1)
    in_xmin = jnp.maximum(gx1, px1)
    in_ymax = jnp.minimum(gy2, py2)
    in_xmax = jnp.minimum(gx2, px2)
    in_w = jnp.maximum(0.0, in_xmax - in_xmin)
    in_h = jnp.maximum(0.0, in_ymax - in_ymin)
    inter = in_w * in_h
    areas = (g_w * g_h) + (p_w * p_h)
    union = areas - inter
    return inter, union


def _split_gt(gt):
    g_cx = gt[:, 0:1]
    g_cy = gt[:, 1:2]
    g_w = gt[:, 2:3]
    g_h = gt[:, 3:4]
    return g_cx, g_cy, g_w, g_h


def _argmax_kernel(gt_ref, px_ref, py_ref, pw_ref, ph_ref, best_ref, m_ref,
                   a_ref, *, blk, num_pr, pb_steps):
    pb = pl.program_id(1)
    gt = gt_ref[0]
    ng = gt.shape[0]
    g_cx, g_cy, g_w, g_h = _split_gt(gt)
    valid = g_cx != -1.0
    inter, union = _inter_union(g_cx, g_cy, g_w, g_h,
                                px_ref[0], py_ref[0], pw_ref[0], ph_ref[0])
    iou = inter / (union + 1e-5)
    iou = jnp.where(valid, iou, 0.0)

    @pl.when(pb == 0)
    def _():
        m_ref[...] = jnp.full((ng, 1), -1.0, jnp.float32)
        a_ref[...] = jnp.zeros((ng, 1), jnp.int32)

    bmax = jnp.max(iou, axis=1, keepdims=True)
    pidx = jax.lax.broadcasted_iota(jnp.int32, iou.shape, 1) + pb * blk
    barg = jnp.min(jnp.where(iou == bmax, pidx, num_pr), axis=1, keepdims=True)
    better = bmax > m_ref[...]
    a_ref[...] = jnp.where(better, barg, a_ref[...])
    m_ref[...] = jnp.where(better, bmax, m_ref[...])

    @pl.when(pb == pb_steps - 1)
    def _():
        best_ref[0] = a_ref[...]


def _assign_kernel(gt_ref, px_ref, py_ref, pw_ref, ph_ref,
                   bx_ref, by_ref, bw_ref, bh_ref, best_ref, packed_ref,
                   *, blk):
    pb = pl.program_id(1)
    gt = gt_ref[0]
    ng = gt.shape[0]
    g_cx, g_cy, g_w, g_h = _split_gt(gt)
    g_cls = gt[:, 4:5]
    g_conf = gt[:, 5:6]
    valid = g_cx != -1.0

    inter, union = _inter_union(g_cx, g_cy, g_w, g_h,
                                px_ref[0], py_ref[0], pw_ref[0], ph_ref[0])
    ue = union + 1e-5
    thr = (inter >= 0.5 * ue) & valid
    ign = (inter >= 0.4 * ue) & (inter < 0.5 * ue) & valid

    pidx = jax.lax.broadcasted_iota(jnp.int32, inter.shape, 1) + pb * blk
    best = best_ref[0]
    is_best = (pidx == best) & (g_conf > 0.0)

    g_iota = jax.lax.broadcasted_iota(jnp.int32, inter.shape, 0)
    score = jnp.where(is_best, g_iota + ng, jnp.where(thr, g_iota, -1))
    smax = jnp.max(score, axis=0, keepdims=True)
    matched = smax >= 0
    sel = (score == smax) & matched
    cls_true = jnp.sum(jnp.where(sel, g_cls, 0.0), axis=0, keepdims=True)
    cls_true = jnp.where(matched, cls_true, float(NC))

    b_cx = bx_ref[0]
    b_cy = by_ref[0]
    b_w = bw_ref[0]
    b_h = bh_ref[0]
    cnt = thr.astype(jnp.float32) + is_best.astype(jnp.float32)
    lgw = jnp.log(jnp.where(valid, g_w, 1.0))
    lgh = jnp.log(jnp.where(valid, g_h, 1.0))
    s_cnt = jnp.sum(cnt, axis=0, keepdims=True)
    s_cx = jnp.sum(cnt * g_cx, axis=0, keepdims=True)
    s_cy = jnp.sum(cnt * g_cy, axis=0, keepdims=True)
    s_lw = jnp.sum(cnt * lgw, axis=0, keepdims=True)
    s_lh = jnp.sum(cnt * lgh, axis=0, keepdims=True)
    l0 = (s_cx - b_cx * s_cnt) / b_w
    l1 = (s_cy - b_cy * s_cnt) / b_h
    l2 = s_lw - s_cnt * jnp.log(b_w)
    l3 = s_lh - s_cnt * jnp.log(b_h)

    bg = (cls_true == float(NC)).astype(jnp.float32)
    ignore_any = jnp.max(ign.astype(jnp.int32), axis=0, keepdims=True) > 0
    amask = jnp.where(ignore_any, -1.0, bg)

    zeros2 = jnp.zeros((2,) + cls_true.shape[1:], jnp.float32)
    packed = jnp.concatenate(
        [cls_true, l0, l1, l2, l3, amask, zeros2], axis=0)
    packed_ref[0] = jnp.transpose(packed)


def _emit_kernel(packed_ref, cls_ref, loc_ref, msk_ref):
    d = packed_ref[0]
    cls_true = d[:, 0:1].astype(jnp.int32)
    c_iota = jax.lax.broadcasted_iota(jnp.int32, (d.shape[0], NC), 1)
    cls_ref[0] = (c_iota == cls_true).astype(jnp.float32)
    loc_ref[0] = d[:, 1:5]
    msk_ref[0] = d[:, 5:6]


@jax.jit
def kernel(gt_boxes, pr_boxes):
    B, NG, _ = gt_boxes.shape
    _, NP, _ = pr_boxes.shape
    blk = 4096
    npad = -NP % blk
    NPP = NP + npad
    pb_steps = NPP // blk

    comp = jnp.pad(pr_boxes, ((0, 0), (0, npad), (0, 0)))
    px = comp[:, :, 0].reshape(B, 1, NPP)
    py = comp[:, :, 1].reshape(B, 1, NPP)
    pw = comp[:, :, 2].reshape(B, 1, NPP)
    ph = comp[:, :, 3].reshape(B, 1, NPP)

    row_spec = lambda f: pl.BlockSpec((1, 1, blk), f)
    b0 = lambda b, p: (0, 0, p)
    bb = lambda b, p: (b, 0, p)

    best = pl.pallas_call(
        functools.partial(_argmax_kernel, blk=blk, num_pr=NPP,
                          pb_steps=pb_steps),
        grid=(B, pb_steps),
        in_specs=[
            pl.BlockSpec((1, NG, 6), lambda b, p: (b, 0, 0)),
            row_spec(b0), row_spec(b0), row_spec(b0), row_spec(b0),
        ],
        out_specs=pl.BlockSpec((1, NG, 1), lambda b, p: (b, 0, 0)),
        out_shape=jax.ShapeDtypeStruct((B, NG, 1), jnp.int32),
        scratch_shapes=[
            pltpu.VMEM((NG, 1), jnp.float32),
            pltpu.VMEM((NG, 1), jnp.int32),
        ],
    )(gt_boxes, px, py, pw, ph)

    packed = pl.pallas_call(
        functools.partial(_assign_kernel, blk=blk),
        grid=(B, pb_steps),
        in_specs=[
            pl.BlockSpec((1, NG, 6), lambda b, p: (b, 0, 0)),
            row_spec(b0), row_spec(b0), row_spec(b0), row_spec(b0),
            row_spec(bb), row_spec(bb), row_spec(bb), row_spec(bb),
            pl.BlockSpec((1, NG, 1), lambda b, p: (b, 0, 0)),
        ],
        out_specs=pl.BlockSpec((1, blk, 8), lambda b, p: (b, p, 0)),
        out_shape=jax.ShapeDtypeStruct((B, NPP, 8), jnp.float32),
    )(gt_boxes, px, py, pw, ph, px, py, pw, ph, best)

    blk3 = 10000
    cls_out, loc_true, amask = pl.pallas_call(
        _emit_kernel,
        grid=(B, NP // blk3),
        in_specs=[pl.BlockSpec((1, blk3, 8), lambda b, p: (b, p, 0))],
        out_specs=[
            pl.BlockSpec((1, blk3, NC), lambda b, p: (b, p, 0)),
            pl.BlockSpec((1, blk3, 4), lambda b, p: (b, p, 0)),
            pl.BlockSpec((1, blk3, 1), lambda b, p: (b, p, 0)),
        ],
        out_shape=[
            jax.ShapeDtypeStruct((B, NP, NC), jnp.float32),
            jax.ShapeDtypeStruct((B, NP, 4), jnp.float32),
            jax.ShapeDtypeStruct((B, NP, 1), jnp.float32),
        ],
    )(packed)

    return (cls_out, loc_true, amask)


# speedup vs baseline: 1.2191x; 1.1251x over previous
import functools

import jax
import jax.numpy as jnp
from jax.experimental import pallas as pl
from jax.experimental.pallas import tpu as pltpu

NC = 80


def _corners(cx, cy, w, h):
    x1 = cx - w / 2
    y1 = cy - h / 2
    x2 = cx + w / 2
    y2 = cy + h / 2
    return y1, x1, y2, x2


def _inter_ue(g_cx, g_cy, g_w, g_h, p_cx, p_cy, p_w, p_h):
    gy1, gx1, gy2, gx2 = _corners(g_cx, g_cy, g_w, g_h)
    py1, px1, py2, px2 = _corners(p_cx, p_cy, p_w, p_h)
    in_ymin = jnp.maximum(gy1, py1)
    in_xmin = jnp.maximum(gx1, px1)
    in_ymax = jnp.minimum(gy2, py2)
    in_xmax = jnp.minimum(gx2, px2)
    in_w = jnp.maximum(0.0, in_xmax - in_xmin)
    in_h = jnp.maximum(0.0, in_ymax - in_ymin)
    inter = in_w * in_h
    pa_eps = p_w * p_h + 1e-5
    ue = ((g_w * g_h) + pa_eps) - inter
    return inter, ue


def _split_gt(gt):
    g_cx = gt[:, 0:1]
    g_cy = gt[:, 1:2]
    g_w = gt[:, 2:3]
    g_h = gt[:, 3:4]
    return g_cx, g_cy, g_w, g_h


def _argmax_kernel(gt_ref, px_ref, py_ref, pw_ref, ph_ref, best_ref, *,
                   num_pr):
    gt = gt_ref[0]
    g_cx, g_cy, g_w, g_h = _split_gt(gt)
    inter, ue = _inter_ue(g_cx, g_cy, g_w, g_h,
                          px_ref[0], py_ref[0], pw_ref[0], ph_ref[0])
    iou = inter / ue
    bmax = jnp.max(iou, axis=1, keepdims=True)
    pidx = jax.lax.broadcasted_iota(jnp.int32, iou.shape, 1)
    barg = jnp.min(jnp.where(iou == bmax, pidx, num_pr), axis=1, keepdims=True)
    best_ref[0] = barg


def _assign_kernel(gt_ref, px_ref, py_ref, pw_ref, ph_ref,
                   bx_ref, by_ref, bw_ref, bh_ref, best_ref, packed_ref,
                   *, blk):
    pb = pl.program_id(1)
    gt = gt_ref[0]
    ng = gt.shape[0]
    g_cx, g_cy, g_w, g_h = _split_gt(gt)
    g_cls = gt[:, 4:5]
    g_conf = gt[:, 5:6]

    inter, ue = _inter_ue(g_cx, g_cy, g_w, g_h,
                          px_ref[0], py_ref[0], pw_ref[0], ph_ref[0])
    thr = (inter + inter) >= ue
    ign = (2.5 * inter >= ue) & (~thr)

    pidx = jax.lax.broadcasted_iota(jnp.int32, (1, blk), 1) + pb * blk
    best = best_ref[0]
    best_x = jnp.where(g_conf > 0.0, best, -7)
    is_best = pidx == best_x

    g_iota = jax.lax.broadcasted_iota(jnp.int32, (ng, 1), 0)
    cls_i = g_cls.astype(jnp.int32)
    v_thr = g_iota * 128 + cls_i
    v_best = v_thr + ng * 128
    score = jnp.where(is_best, v_best, jnp.where(thr, v_thr, -1))
    smax = jnp.max(score, axis=0, keepdims=True)
    matched = smax >= 0
    cls_true = jnp.where(matched, smax & 127, NC).astype(jnp.float32)

    cnt = thr.astype(jnp.float32) + is_best.astype(jnp.float32)
    lgw = jnp.log(jnp.where(g_w > 0.0, g_w, 1.0))
    lgh = jnp.log(jnp.where(g_h > 0.0, g_h, 1.0))
    wmat = jnp.transpose(jnp.concatenate(
        [jnp.ones_like(g_cx), g_cx, g_cy, lgw, lgh], axis=1))
    sums = jax.lax.dot_general(
        wmat, cnt, (((1,), (0,)), ((), ())),
        preferred_element_type=jnp.float32)
    s_cnt = sums[0:1, :]
    s_cx = sums[1:2, :]
    s_cy = sums[2:3, :]
    s_lw = sums[3:4, :]
    s_lh = sums[4:5, :]
    b_cx = bx_ref[0]
    b_cy = by_ref[0]
    b_w = bw_ref[0]
    b_h = bh_ref[0]
    l0 = (s_cx - b_cx * s_cnt) / b_w
    l1 = (s_cy - b_cy * s_cnt) / b_h
    l2 = s_lw - s_cnt * jnp.log(b_w)
    l3 = s_lh - s_cnt * jnp.log(b_h)

    bg = (~matched).astype(jnp.float32)
    ignore_any = jnp.max(ign.astype(jnp.int32), axis=0, keepdims=True) > 0
    amask = jnp.where(ignore_any, -1.0, bg)

    zeros2 = jnp.zeros((2, blk), jnp.float32)
    packed = jnp.concatenate(
        [cls_true, l0, l1, l2, l3, amask, zeros2], axis=0)
    packed_ref[0] = jnp.transpose(packed)


def _emit_kernel(packed_ref, cls_ref, loc_ref, msk_ref):
    d = packed_ref[0]
    cls_true = d[:, 0:1].astype(jnp.int32)
    c_iota = jax.lax.broadcasted_iota(jnp.int32, (d.shape[0], NC), 1)
    cls_ref[0] = (c_iota == cls_true).astype(jnp.float32)
    loc_ref[0] = d[:, 1:5]
    msk_ref[0] = d[:, 5:6]


@jax.jit
def kernel(gt_boxes, pr_boxes):
    B, NG, _ = gt_boxes.shape
    _, NP, _ = pr_boxes.shape
    blk = 4096
    npad = -NP % blk
    NPP = NP + npad
    pb_steps = NPP // blk

    gt_valid = gt_boxes[:, :, 0:1] != -1.0
    gt_clean = jnp.where(gt_valid, gt_boxes, jnp.zeros((), jnp.float32))
    gt_clean = jnp.concatenate([gt_clean[:, :, :5], gt_boxes[:, :, 5:6]],
                               axis=-1)

    comp = jnp.pad(pr_boxes, ((0, 0), (0, npad), (0, 0)))
    px = comp[:, :, 0].reshape(B, 1, NPP)
    py = comp[:, :, 1].reshape(B, 1, NPP)
    pw = comp[:, :, 2].reshape(B, 1, NPP)
    ph = comp[:, :, 3].reshape(B, 1, NPP)

    full_spec = lambda: pl.BlockSpec((1, 1, NPP), lambda b: (0, 0, 0))
    row_spec = lambda f: pl.BlockSpec((1, 1, blk), f)
    b0 = lambda b, p: (0, 0, p)
    bb = lambda b, p: (b, 0, p)

    best = pl.pallas_call(
        functools.partial(_argmax_kernel, num_pr=NPP),
        grid=(B,),
        in_specs=[
            pl.BlockSpec((1, NG, 6), lambda b: (b, 0, 0)),
            full_spec(), full_spec(), full_spec(), full_spec(),
        ],
        out_specs=pl.BlockSpec((1, NG, 1), lambda b: (b, 0, 0)),
        out_shape=jax.ShapeDtypeStruct((B, NG, 1), jnp.int32),
    )(gt_clean, px, py, pw, ph)

    packed = pl.pallas_call(
        functools.partial(_assign_kernel, blk=blk),
        grid=(B, pb_steps),
        in_specs=[
            pl.BlockSpec((1, NG, 6), lambda b, p: (b, 0, 0)),
            row_spec(b0), row_spec(b0), row_spec(b0), row_spec(b0),
            row_spec(bb), row_spec(bb), row_spec(bb), row_spec(bb),
            pl.BlockSpec((1, NG, 1), lambda b, p: (b, 0, 0)),
        ],
        out_specs=pl.BlockSpec((1, blk, 8), lambda b, p: (b, p, 0)),
        out_shape=jax.ShapeDtypeStruct((B, NPP, 8), jnp.float32),
    )(gt_clean, px, py, pw, ph, px, py, pw, ph, best)

    blk3 = 10000
    cls_out, loc_true, amask = pl.pallas_call(
        _emit_kernel,
        grid=(B, NP // blk3),
        in_specs=[pl.BlockSpec((1, blk3, 8), lambda b, p: (b, p, 0))],
        out_specs=[
            pl.BlockSpec((1, blk3, NC), lambda b, p: (b, p, 0)),
            pl.BlockSpec((1, blk3, 4), lambda b, p: (b, p, 0)),
            pl.BlockSpec((1, blk3, 1), lambda b, p: (b, p, 0)),
        ],
        out_shape=[
            jax.ShapeDtypeStruct((B, NP, NC), jnp.float32),
            jax.ShapeDtypeStruct((B, NP, 4), jnp.float32),
            jax.ShapeDtypeStruct((B, NP, 1), jnp.float32),
        ],
    )(packed)

    return (cls_out, loc_true, amask)
